# Initial kernel scaffold; baseline (speedup 1.0000x reference)
import jax
import jax.numpy as jnp
from jax.experimental import pallas as pl


def kernel(pc, feature, W, b):
    raise NotImplementedError("write your pallas kernel here")


# speedup vs baseline: 1.0010x; 1.0010x over previous
import jax
import jax.numpy as jnp
from jax.experimental import pallas as pl
from jax.experimental.pallas import tpu as pltpu

K = 16
NT = 512


def _conv_body(w_ref, b_ref, f_ref, o_ref):
    acc = jnp.dot(w_ref[...], f_ref[0], preferred_element_type=jnp.float32)
    o_ref[0] = jnp.maximum(acc + b_ref[...], 0.0)


def _conv_all(feature, W, b):
    B, C, N = feature.shape
    CO = W.shape[0]
    grid = (B, N // NT)
    return pl.pallas_call(
        _conv_body,
        grid=grid,
        in_specs=[
            pl.BlockSpec((CO, C), lambda bi, ni: (0, 0)),
            pl.BlockSpec((CO, 1), lambda bi, ni: (0, 0)),
            pl.BlockSpec((1, C, NT), lambda bi, ni: (bi, 0, ni)),
        ],
        out_specs=pl.BlockSpec((1, CO, NT), lambda bi, ni: (bi, 0, ni)),
        out_shape=jax.ShapeDtypeStruct((B, CO, N), jnp.float32),
    )(W, b[:, None], feature)


def _fps(pts, m):
    n = pts.shape[0]

    def body(i, carry):
        dists, last, idxs = carry
        d = jnp.sum((pts - pts[last]) ** 2, axis=1)
        dists = jnp.minimum(dists, d)
        nxt = jnp.argmax(dists).astype(jnp.int32)
        idxs = idxs.at[i].set(nxt)
        return (dists, nxt, idxs)

    idxs = jnp.zeros((m,), dtype=jnp.int32)
    dists = jnp.full((n,), 1e10, dtype=jnp.float32)
    dists, last, idxs = jax.lax.fori_loop(1, m, body, (dists, jnp.int32(0), idxs))
    return idxs


def kernel(pc, feature, W, b):
    Bb, C, Nn = feature.shape
    m = Nn // 2
    pc_t = jnp.transpose(pc, (0, 2, 1))
    fps_idx = jax.vmap(lambda p: _fps(p, m))(pc_t)
    fps_pc = jnp.take_along_axis(pc_t, fps_idx[..., None], axis=1)

    Yt = _conv_all(feature, W, b)
    Y = jnp.transpose(Yt, (0, 2, 1))

    def knn_max(args):
        fp, p, y = args
        d = (jnp.sum(fp * fp, axis=1)[:, None] + jnp.sum(p * p, axis=1)[None, :]
             - 2.0 * (fp @ p.T))
        _, nidx = jax.lax.top_k(-d, K)
        return jnp.max(jnp.take(y, nidx, axis=0), axis=1)

    out = jax.lax.map(knn_max, (fps_pc, pc_t, Y))
    fps_feature = jnp.transpose(out, (0, 2, 1))
    fps_pc_out = jnp.transpose(fps_pc, (0, 2, 1))
    return (fps_pc_out, fps_feature, fps_idx.astype(jnp.int64))


# speedup vs baseline: 2.9775x; 2.9746x over previous
import jax
import jax.numpy as jnp
from jax.experimental import pallas as pl
from jax.experimental.pallas import tpu as pltpu

K = 16
NT = 512


def _conv_body(w_ref, b_ref, f_ref, o_ref):
    acc = jnp.dot(w_ref[...], f_ref[0], preferred_element_type=jnp.float32)
    o_ref[0] = jnp.maximum(acc + b_ref[...], 0.0)


def _conv_all(feature, W, b):
    B, C, N = feature.shape
    CO = W.shape[0]
    grid = (B, N // NT)
    return pl.pallas_call(
        _conv_body,
        grid=grid,
        in_specs=[
            pl.BlockSpec((CO, C), lambda bi, ni: (0, 0)),
            pl.BlockSpec((CO, 1), lambda bi, ni: (0, 0)),
            pl.BlockSpec((1, C, NT), lambda bi, ni: (bi, 0, ni)),
        ],
        out_specs=pl.BlockSpec((1, CO, NT), lambda bi, ni: (bi, 0, ni)),
        out_shape=jax.ShapeDtypeStruct((B, CO, N), jnp.float32),
    )(W, b[:, None], feature)


def _fps_body(pc_ref, idx_ref, dists_ref):
    B, _, N = pc_ref.shape
    M = idx_ref.shape[1]
    X = pc_ref[:, 0, :]
    Y = pc_ref[:, 1, :]
    Z = pc_ref[:, 2, :]
    lane = jax.lax.broadcasted_iota(jnp.int32, (B, N), 1)
    lane128 = jax.lax.broadcasted_iota(jnp.int32, (B, 128), 1)
    dists_ref[...] = jnp.full((B, N), 1e10, jnp.float32)

    def body(j, carry):
        xl, yl, zl, picks = carry
        dx = X - xl
        dy = Y - yl
        dz = Z - zl
        d = (dx * dx + dz * dz) + dy * dy
        dists = jnp.minimum(dists_ref[...], d)
        dists_ref[...] = dists
        mx = jnp.max(dists, axis=1, keepdims=True)
        eq = dists == mx
        nxt = jnp.min(jnp.where(eq, lane, N), axis=1, keepdims=True)
        picks = jnp.where(lane128 == j, nxt, picks)
        sel = lane == nxt
        ninf = jnp.float32(-jnp.inf)
        xn = jnp.max(jnp.where(sel, X, ninf), axis=1, keepdims=True)
        yn = jnp.max(jnp.where(sel, Y, ninf), axis=1, keepdims=True)
        zn = jnp.max(jnp.where(sel, Z, ninf), axis=1, keepdims=True)
        return (xn, yn, zn, picks)

    carry = (X[:, 0:1], Y[:, 0:1], Z[:, 0:1],
             jnp.zeros((B, 128), jnp.int32))
    for blk in range(M // 128):
        lo = 1 if blk == 0 else 0
        carry = jax.lax.fori_loop(lo, 128, body, carry)
        idx_ref[:, blk * 128:(blk + 1) * 128] = carry[3]
        carry = (carry[0], carry[1], carry[2], jnp.zeros((B, 128), jnp.int32))


def _fps_all(pc, m):
    B, _, N = pc.shape
    return pl.pallas_call(
        _fps_body,
        out_shape=jax.ShapeDtypeStruct((B, m), jnp.int32),
        scratch_shapes=[pltpu.VMEM((B, N), jnp.float32)],
    )(pc)


def kernel(pc, feature, W, b):
    Bb, C, Nn = feature.shape
    m = Nn // 2
    pc_t = jnp.transpose(pc, (0, 2, 1))
    fps_idx = _fps_all(pc, m)
    fps_pc = jnp.take_along_axis(pc_t, fps_idx[..., None], axis=1)

    Yt = _conv_all(feature, W, b)
    Y = jnp.transpose(Yt, (0, 2, 1))

    def knn_max(args):
        fp, p, y = args
        d = (jnp.sum(fp * fp, axis=1)[:, None] + jnp.sum(p * p, axis=1)[None, :]
             - 2.0 * (fp @ p.T))
        _, nidx = jax.lax.top_k(-d, K)
        return jnp.max(jnp.take(y, nidx, axis=0), axis=1)

    out = jax.lax.map(knn_max, (fps_pc, pc_t, Y))
    fps_feature = jnp.transpose(out, (0, 2, 1))
    fps_pc_out = jnp.transpose(fps_pc, (0, 2, 1))
    return (fps_pc_out, fps_feature, fps_idx.astype(jnp.int64))


# speedup vs baseline: 12.8081x; 4.3016x over previous
import functools

import jax
import jax.numpy as jnp
from jax import lax
from jax.experimental import pallas as pl
from jax.experimental.pallas import tpu as pltpu
from jax.experimental.pallas import tpu_sc as plsc

K = 16
NT = 512
QT = 128


def _conv_body(w_ref, b_ref, f_ref, o_ref):
    acc = jnp.dot(w_ref[...], f_ref[0], preferred_element_type=jnp.float32)
    o_ref[0] = jnp.maximum(acc + b_ref[...], 0.0)


def _conv_all(feature, W, b):
    B, C, N = feature.shape
    CO = W.shape[0]
    grid = (B, N // NT)
    return pl.pallas_call(
        _conv_body,
        grid=grid,
        in_specs=[
            pl.BlockSpec((CO, C), lambda bi, ni: (0, 0)),
            pl.BlockSpec((CO, 1), lambda bi, ni: (0, 0)),
            pl.BlockSpec((1, C, NT), lambda bi, ni: (bi, 0, ni)),
        ],
        out_specs=pl.BlockSpec((1, CO, NT), lambda bi, ni: (bi, 0, ni)),
        out_shape=jax.ShapeDtypeStruct((B, CO, N), jnp.float32),
    )(W, b[:, None], feature)


def _fps_body(pc_ref, idx_ref, dists_ref):
    B, _, N = pc_ref.shape
    M = idx_ref.shape[1]
    X = pc_ref[:, 0, :]
    Y = pc_ref[:, 1, :]
    Z = pc_ref[:, 2, :]
    lane = jax.lax.broadcasted_iota(jnp.int32, (B, N), 1)
    lane128 = jax.lax.broadcasted_iota(jnp.int32, (B, 128), 1)
    dists_ref[...] = jnp.full((B, N), 1e10, jnp.float32)

    def body(j, carry):
        xl, yl, zl, picks = carry
        dx = X - xl
        dy = Y - yl
        dz = Z - zl
        d = (dx * dx + dz * dz) + dy * dy
        dists = jnp.minimum(dists_ref[...], d)
        dists_ref[...] = dists
        mx = jnp.max(dists, axis=1, keepdims=True)
        eq = dists == mx
        nxt = jnp.min(jnp.where(eq, lane, N), axis=1, keepdims=True)
        picks = jnp.where(lane128 == j, nxt, picks)
        sel = lane == nxt
        ninf = jnp.float32(-jnp.inf)
        xn = jnp.max(jnp.where(sel, X, ninf), axis=1, keepdims=True)
        yn = jnp.max(jnp.where(sel, Y, ninf), axis=1, keepdims=True)
        zn = jnp.max(jnp.where(sel, Z, ninf), axis=1, keepdims=True)
        return (xn, yn, zn, picks)

    carry = (X[:, 0:1], Y[:, 0:1], Z[:, 0:1],
             jnp.zeros((B, 128), jnp.int32))
    for blk in range(M // 128):
        lo = 1 if blk == 0 else 0
        carry = jax.lax.fori_loop(lo, 128, body, carry)
        idx_ref[:, blk * 128:(blk + 1) * 128] = carry[3]
        carry = (carry[0], carry[1], carry[2], jnp.zeros((B, 128), jnp.int32))


def _fps_all(pc, m):
    B, _, N = pc.shape
    return pl.pallas_call(
        _fps_body,
        out_shape=jax.ShapeDtypeStruct((B, m), jnp.int32),
        scratch_shapes=[pltpu.VMEM((B, N), jnp.float32)],
    )(pc)


def _knn_body(fq_ref, p_ref, nidx_ref):
    _, _, N = p_ref.shape
    fq = fq_ref[0]
    p = p_ref[0]
    qx = fq[:, 0:1]
    qy = fq[:, 1:2]
    qz = fq[:, 2:3]
    qn = (qx * qx + qz * qz) + qy * qy
    px = p[0:1, :]
    py = p[1:2, :]
    pz = p[2:3, :]
    pn = (px * px + pz * pz) + py * py
    g = jnp.dot(fq, p, preferred_element_type=jnp.float32)
    d = (qn + pn) - 2.0 * g
    lane = jax.lax.broadcasted_iota(jnp.int32, (QT, N), 1)
    lane16 = jax.lax.broadcasted_iota(jnp.int32, (QT, K), 1)
    inf = jnp.float32(jnp.inf)
    acc = jnp.zeros((QT, K), jnp.int32)
    for j in range(K):
        mn = jnp.min(d, axis=1, keepdims=True)
        idx = jnp.min(jnp.where(d == mn, lane, N), axis=1, keepdims=True)
        acc = jnp.where(lane16 == j, idx, acc)
        d = jnp.where(lane == idx, inf, d)
    nidx_ref[0] = acc


def _knn_all(fps_pc, pc):
    B, m, _ = fps_pc.shape
    N = pc.shape[2]
    grid = (B, m // QT)
    return pl.pallas_call(
        _knn_body,
        grid=grid,
        in_specs=[
            pl.BlockSpec((1, QT, 3), lambda bi, qi: (bi, qi, 0)),
            pl.BlockSpec((1, 3, N), lambda bi, qi: (bi, 0, 0)),
        ],
        out_specs=pl.BlockSpec((1, QT, K), lambda bi, qi: (bi, qi, 0)),
        out_shape=jax.ShapeDtypeStruct((B, m, K), jnp.int32),
    )(fps_pc, pc)


def _gathermax_all(Y2d, gidx, CO):
    R, D = Y2d.shape
    Qfull = gidx.shape[0]
    Q = Qfull // 2
    info = plsc.get_sparse_core_info()
    NW = info.num_cores * info.num_subcores
    qpw = Q // NW
    mesh = plsc.VectorSubcoreMesh(core_axis_name="c", subcore_axis_name="s")

    CH = qpw // 2

    @functools.partial(
        pl.kernel, mesh=mesh,
        out_type=jax.ShapeDtypeStruct((Q, CO), jnp.float32),
        scratch_types=[
            pltpu.VMEM((qpw, K), jnp.int32),
            pltpu.VMEM((K, D), jnp.float32),
            pltpu.VMEM((CH, CO), jnp.float32),
            pltpu.SemaphoreType.DMA,
        ],
    )
    def k(y_hbm, idx_hbm, out_hbm, idx_v, rows_v, out_v, sem):
        wid = lax.axis_index("s") * info.num_cores + lax.axis_index("c")
        base = wid * qpw
        pltpu.sync_copy(idx_hbm.at[pl.ds(base, qpw)], idx_v)

        for h in range(2):
            def body(q, _):
                idx_vec = idx_v[h * CH + q]
                pltpu.async_copy(y_hbm.at[idx_vec], rows_v, sem).wait()
                for c in range(CO // 16):
                    acc = rows_v[0, pl.ds(c * 16, 16)]
                    for r in range(1, K):
                        acc = jnp.maximum(acc, rows_v[r, pl.ds(c * 16, 16)])
                    out_v[q, pl.ds(c * 16, 16)] = acc
                return 0

            lax.fori_loop(0, CH, body, 0)
            pltpu.sync_copy(out_v, out_hbm.at[pl.ds(base + h * CH, CH)])

    return jnp.concatenate([k(Y2d, gidx[:Q]), k(Y2d, gidx[Q:])], axis=0)


def kernel(pc, feature, W, b):
    Bb, C, Nn = feature.shape
    m = Nn // 2
    pc_t = jnp.transpose(pc, (0, 2, 1))
    fps_idx = _fps_all(pc, m)
    fps_pc = jnp.take_along_axis(pc_t, fps_idx[..., None], axis=1)

    Yt = _conv_all(feature, W, b)
    Y = jnp.transpose(Yt, (0, 2, 1))

    nidx = _knn_all(fps_pc, pc)
    gidx = (nidx + (jnp.arange(Bb, dtype=jnp.int32) * Nn)[:, None, None])
    CO = W.shape[0]
    Yp = jnp.pad(Y, ((0, 0), (0, 0), (0, 128 - CO)))
    out2d = _gathermax_all(Yp.reshape(Bb * Nn, 128),
                           gidx.reshape(Bb * m, K), CO)
    out = out2d.reshape(Bb, m, CO)
    fps_feature = jnp.transpose(out, (0, 2, 1))
    fps_pc_out = jnp.transpose(fps_pc, (0, 2, 1))
    return (fps_pc_out, fps_feature, fps_idx.astype(jnp.int64))


# speedup vs baseline: 12.8404x; 1.0025x over previous
import functools

import jax
import jax.numpy as jnp
from jax import lax
from jax.experimental import pallas as pl
from jax.experimental.pallas import tpu as pltpu
from jax.experimental.pallas import tpu_sc as plsc

K = 16
NT = 512
QT = 128


def _conv_body(w_ref, b_ref, f_ref, o_ref):
    acc = jnp.dot(w_ref[...], f_ref[0], preferred_element_type=jnp.float32)
    y = jnp.maximum(acc + b_ref[...], 0.0)
    yt = y.T
    o_ref[0] = jnp.concatenate(
        [yt, jnp.zeros((yt.shape[0], 128 - yt.shape[1]), jnp.float32)], axis=1)


def _conv_all(feature, W, b):
    B, C, N = feature.shape
    CO = W.shape[0]
    grid = (B, N // NT)
    return pl.pallas_call(
        _conv_body,
        grid=grid,
        in_specs=[
            pl.BlockSpec((CO, C), lambda bi, ni: (0, 0)),
            pl.BlockSpec((CO, 1), lambda bi, ni: (0, 0)),
            pl.BlockSpec((1, C, NT), lambda bi, ni: (bi, 0, ni)),
        ],
        out_specs=pl.BlockSpec((1, NT, 128), lambda bi, ni: (bi, ni, 0)),
        out_shape=jax.ShapeDtypeStruct((B, N, 128), jnp.float32),
    )(W, b[:, None], feature)


def _fps_body(pc_ref, idx_ref, dists_ref):
    B, _, N = pc_ref.shape
    M = idx_ref.shape[1]
    X = pc_ref[:, 0, :]
    Y = pc_ref[:, 1, :]
    Z = pc_ref[:, 2, :]
    lane = jax.lax.broadcasted_iota(jnp.int32, (B, N), 1)
    lane128 = jax.lax.broadcasted_iota(jnp.int32, (B, 128), 1)
    dists_ref[...] = jnp.full((B, N), 1e10, jnp.float32)

    def body(j, carry):
        xl, yl, zl, picks = carry
        dx = X - xl
        dy = Y - yl
        dz = Z - zl
        d = (dx * dx + dz * dz) + dy * dy
        dists = jnp.minimum(dists_ref[...], d)
        dists_ref[...] = dists
        mx = jnp.max(dists, axis=1, keepdims=True)
        eq = dists == mx
        nxt = jnp.min(jnp.where(eq, lane, N), axis=1, keepdims=True)
        picks = jnp.where(lane128 == j, nxt, picks)
        sel = lane == nxt
        ninf = jnp.float32(-jnp.inf)
        xn = jnp.max(jnp.where(sel, X, ninf), axis=1, keepdims=True)
        yn = jnp.max(jnp.where(sel, Y, ninf), axis=1, keepdims=True)
        zn = jnp.max(jnp.where(sel, Z, ninf), axis=1, keepdims=True)
        return (xn, yn, zn, picks)

    carry = (X[:, 0:1], Y[:, 0:1], Z[:, 0:1],
             jnp.zeros((B, 128), jnp.int32))
    for blk in range(M // 128):
        lo = 1 if blk == 0 else 0
        carry = jax.lax.fori_loop(lo, 128, body, carry)
        idx_ref[:, blk * 128:(blk + 1) * 128] = carry[3]
        carry = (carry[0], carry[1], carry[2], jnp.zeros((B, 128), jnp.int32))


def _fps_all(pc, m):
    B, _, N = pc.shape
    return pl.pallas_call(
        _fps_body,
        out_shape=jax.ShapeDtypeStruct((B, m), jnp.int32),
        scratch_shapes=[pltpu.VMEM((B, N), jnp.float32)],
    )(pc)


def _knn_body(fq_ref, p_ref, nidx_ref):
    _, _, N = p_ref.shape
    fq = fq_ref[0]
    p = p_ref[0]
    qx = fq[:, 0:1]
    qy = fq[:, 1:2]
    qz = fq[:, 2:3]
    qn = (qx * qx + qz * qz) + qy * qy
    px = p[0:1, :]
    py = p[1:2, :]
    pz = p[2:3, :]
    pn = (px * px + pz * pz) + py * py
    g = jnp.dot(fq, p, preferred_element_type=jnp.float32)
    d = (qn + pn) - 2.0 * g
    lane = jax.lax.broadcasted_iota(jnp.int32, (QT, N), 1)
    lane16 = jax.lax.broadcasted_iota(jnp.int32, (QT, K), 1)
    inf = jnp.float32(jnp.inf)
    acc = jnp.zeros((QT, K), jnp.int32)
    for j in range(K):
        mn = jnp.min(d, axis=1, keepdims=True)
        idx = jnp.min(jnp.where(d == mn, lane, N), axis=1, keepdims=True)
        acc = jnp.where(lane16 == j, idx, acc)
        d = jnp.where(lane == idx, inf, d)
    nidx_ref[0] = acc


def _knn_all(fps_pc, pc):
    B, m, _ = fps_pc.shape
    N = pc.shape[2]
    grid = (B, m // QT)
    return pl.pallas_call(
        _knn_body,
        grid=grid,
        in_specs=[
            pl.BlockSpec((1, QT, 3), lambda bi, qi: (bi, qi, 0)),
            pl.BlockSpec((1, 3, N), lambda bi, qi: (bi, 0, 0)),
        ],
        out_specs=pl.BlockSpec((1, QT, K), lambda bi, qi: (bi, qi, 0)),
        out_shape=jax.ShapeDtypeStruct((B, m, K), jnp.int32),
    )(fps_pc, pc)


def _gathermax_all(Y2d, gidx, CO):
    R, D = Y2d.shape
    Qfull = gidx.shape[0]
    Q = Qfull // 2
    info = plsc.get_sparse_core_info()
    NW = info.num_cores * info.num_subcores
    qpw = Q // NW
    mesh = plsc.VectorSubcoreMesh(core_axis_name="c", subcore_axis_name="s")

    CH = qpw // 2

    @functools.partial(
        pl.kernel, mesh=mesh,
        out_type=jax.ShapeDtypeStruct((Q, CO), jnp.float32),
        scratch_types=[
            pltpu.VMEM((qpw, K), jnp.int32),
            pltpu.VMEM((K, D), jnp.float32),
            pltpu.VMEM((CH, CO), jnp.float32),
            pltpu.SemaphoreType.DMA,
        ],
    )
    def k(y_hbm, idx_hbm, out_hbm, idx_v, rows_v, out_v, sem):
        wid = lax.axis_index("s") * info.num_cores + lax.axis_index("c")
        base = wid * qpw
        pltpu.sync_copy(idx_hbm.at[pl.ds(base, qpw)], idx_v)

        for h in range(2):
            def body(q, _):
                idx_vec = idx_v[h * CH + q]
                pltpu.async_copy(y_hbm.at[idx_vec], rows_v, sem).wait()
                for c in range(CO // 16):
                    acc = rows_v[0, pl.ds(c * 16, 16)]
                    for r in range(1, K):
                        acc = jnp.maximum(acc, rows_v[r, pl.ds(c * 16, 16)])
                    out_v[q, pl.ds(c * 16, 16)] = acc
                return 0

            lax.fori_loop(0, CH, body, 0)
            pltpu.sync_copy(out_v, out_hbm.at[pl.ds(base + h * CH, CH)])

    return jnp.concatenate([k(Y2d, gidx[:Q]), k(Y2d, gidx[Q:])], axis=0)


def kernel(pc, feature, W, b):
    Bb, C, Nn = feature.shape
    m = Nn // 2
    pc_t = jnp.transpose(pc, (0, 2, 1))
    fps_idx = _fps_all(pc, m)
    fps_pc = jnp.take_along_axis(pc_t, fps_idx[..., None], axis=1)

    Yp = _conv_all(feature, W, b)

    nidx = _knn_all(fps_pc, pc)
    gidx = (nidx + (jnp.arange(Bb, dtype=jnp.int32) * Nn)[:, None, None])
    CO = W.shape[0]
    out2d = _gathermax_all(Yp.reshape(Bb * Nn, 128),
                           gidx.reshape(Bb * m, K), CO)
    out = out2d.reshape(Bb, m, CO)
    fps_feature = jnp.transpose(out, (0, 2, 1))
    fps_pc_out = jnp.transpose(fps_pc, (0, 2, 1))
    return (fps_pc_out, fps_feature, fps_idx.astype(jnp.int64))
